# baseline (speedup 1.0000x reference)
import functools

import jax
import jax.numpy as jnp
import numpy as np
from jax.experimental import pallas as pl
from jax.experimental.pallas import tpu as pltpu

_TIMESTEPS = 100
_ROWS_PER_BLOCK = 512


def _threefry_bits(e):
    ks0 = jnp.uint32(0)
    ks1 = jnp.uint32(42)
    ks2 = jnp.uint32(0x1BD11BDA ^ 42)
    rot1 = (13, 15, 26, 6)
    rot2 = (17, 29, 16, 24)

    x0 = jnp.zeros_like(e)
    x1 = e + ks1

    def rounds(x0, x1, rots):
        for r in rots:
            x0 = x0 + x1
            x1 = ((x1 << jnp.uint32(r)) | (x1 >> jnp.uint32(32 - r))) ^ x0
        return x0, x1

    x0, x1 = rounds(x0, x1, rot1)
    x0 = x0 + ks1
    x1 = x1 + (ks2 + jnp.uint32(1))
    x0, x1 = rounds(x0, x1, rot2)
    x0 = x0 + ks2
    x1 = x1 + (ks0 + jnp.uint32(2))
    x0, x1 = rounds(x0, x1, rot1)
    x0 = x0 + ks0
    x1 = x1 + (ks1 + jnp.uint32(3))
    x0, x1 = rounds(x0, x1, rot2)
    x0 = x0 + ks1
    x1 = x1 + (ks2 + jnp.uint32(4))
    x0, x1 = rounds(x0, x1, rot1)
    x0 = x0 + ks2
    x1 = x1 + (ks0 + jnp.uint32(5))
    return x0 ^ x1


def _body(m_ref, x_ref, xt_ref, ign_ref):
    blk = pl.program_id(0)
    r, n = x_ref.shape
    row = jnp.uint32(blk * r) + jax.lax.broadcasted_iota(jnp.uint32, (r, n), 0)
    col = jax.lax.broadcasted_iota(jnp.uint32, (r, n), 1)
    e = row * jnp.uint32(n) + col
    bits = _threefry_bits(e)
    mask = (bits >> jnp.uint32(9)) < m_ref[...].astype(jnp.uint32)

    x = x_ref[...]
    xt_ref[...] = jnp.where(mask, jnp.int32(21), x)
    ign_ref[...] = jnp.where(mask, x, jnp.int32(-1))


@functools.partial(jax.jit)
def kernel(x_0, t):
    b, n = x_0.shape
    total = b * n
    w = 128
    rows = total // w
    x32 = x_0.astype(jnp.int32).reshape(rows, w)
    p = t.astype(jnp.float32) / _TIMESTEPS
    m = jnp.ceil(p.astype(jnp.float64) * (2.0 ** 23)).astype(jnp.int32)
    mwide = jnp.repeat(m, n, total_repeat_length=total).reshape(rows, w)

    r = _ROWS_PER_BLOCK
    grid = (rows // r,)
    _imap = lambda i: (i, np.int32(0))
    xt32, ign32 = pl.pallas_call(
        _body,
        grid=grid,
        in_specs=[
            pl.BlockSpec((r, w), _imap),
            pl.BlockSpec((r, w), _imap),
        ],
        out_specs=[
            pl.BlockSpec((r, w), _imap),
            pl.BlockSpec((r, w), _imap),
        ],
        out_shape=[
            jax.ShapeDtypeStruct((rows, w), jnp.int32),
            jax.ShapeDtypeStruct((rows, w), jnp.int32),
        ],
        compiler_params=pltpu.CompilerParams(
            dimension_semantics=("parallel",),
        ),
    )(mwide, x32)
    x_t = xt32.reshape(b, n).astype(jnp.int64)
    x_0_ignore = ign32.reshape(b, n).astype(jnp.int64)
    return (x_t, x_0_ignore)


# speedup vs baseline: 36.4546x; 36.4546x over previous
import functools

import jax
import jax.numpy as jnp
import numpy as np
from jax.experimental import pallas as pl
from jax.experimental.pallas import tpu as pltpu

_TIMESTEPS = 100
_RES = 200
_ROWS_PER_BLOCK = 512


def _threefry_bits(e):
    ks0 = jnp.uint32(0)
    ks1 = jnp.uint32(42)
    ks2 = jnp.uint32(0x1BD11BDA ^ 42)
    rot1 = (13, 15, 26, 6)
    rot2 = (17, 29, 16, 24)

    x0 = jnp.zeros_like(e)
    x1 = e + ks1

    def rounds(x0, x1, rots):
        for r in rots:
            x0 = x0 + x1
            x1 = ((x1 << jnp.uint32(r)) | (x1 >> jnp.uint32(32 - r))) ^ x0
        return x0, x1

    x0, x1 = rounds(x0, x1, rot1)
    x0 = x0 + ks1
    x1 = x1 + (ks2 + jnp.uint32(1))
    x0, x1 = rounds(x0, x1, rot2)
    x0 = x0 + ks2
    x1 = x1 + (ks0 + jnp.uint32(2))
    x0, x1 = rounds(x0, x1, rot1)
    x0 = x0 + ks0
    x1 = x1 + (ks1 + jnp.uint32(3))
    x0, x1 = rounds(x0, x1, rot2)
    x0 = x0 + ks1
    x1 = x1 + (ks2 + jnp.uint32(4))
    x0, x1 = rounds(x0, x1, rot1)
    x0 = x0 + ks2
    x1 = x1 + (ks0 + jnp.uint32(5))
    return x0 ^ x1


def _body(m_ref, x_ref, xt_ref, ign_ref):
    blk = pl.program_id(0)
    r = x_ref.shape[0]
    n = x_ref.shape[1]
    row = jnp.uint32(blk * r) + jax.lax.broadcasted_iota(jnp.uint32, (r, n), 0)
    col = jax.lax.broadcasted_iota(jnp.uint32, (r, n), 1)
    e = row * jnp.uint32(n) + col
    bits = _threefry_bits(e)
    mask = (bits >> jnp.uint32(9)) < m_ref[...].astype(jnp.uint32)

    x = x_ref[...]
    xt_ref[...] = jnp.where(mask, jnp.int32(21), x)
    ign_ref[...] = jnp.where(mask, x, jnp.int32(-1))


@functools.partial(jax.jit)
def kernel(x_0, t):
    b, n = x_0.shape
    x32 = x_0.astype(jnp.int32)
    p = t.astype(jnp.float32) / _TIMESTEPS
    m = jnp.ceil(p.astype(jnp.float64) * (2.0 ** 23)).astype(jnp.int32)
    m = m.reshape(b, 1)

    r = _ROWS_PER_BLOCK
    grid = (b // r,)
    _imap = lambda i: (i, np.int32(0))
    xt32, ign32 = pl.pallas_call(
        _body,
        grid=grid,
        in_specs=[
            pl.BlockSpec((r, 1), _imap),
            pl.BlockSpec((r, n), _imap),
        ],
        out_specs=[
            pl.BlockSpec((r, n), _imap),
            pl.BlockSpec((r, n), _imap),
        ],
        out_shape=[
            jax.ShapeDtypeStruct((b, n), jnp.int32),
            jax.ShapeDtypeStruct((b, n), jnp.int32),
        ],
        compiler_params=pltpu.CompilerParams(
            dimension_semantics=("parallel",),
        ),
    )(m, x32)
    return (xt32.astype(jnp.int64), ign32.astype(jnp.int64))


# speedup vs baseline: 36.8889x; 1.0119x over previous
import functools

import jax
import jax.numpy as jnp
import numpy as np
from jax.experimental import pallas as pl
from jax.experimental.pallas import tpu as pltpu

_TIMESTEPS = 100
_RES = 200
_ROWS_PER_BLOCK = 256


def _threefry_bits(e):
    ks0 = jnp.uint32(0)
    ks1 = jnp.uint32(42)
    ks2 = jnp.uint32(0x1BD11BDA ^ 42)
    rot1 = (13, 15, 26, 6)
    rot2 = (17, 29, 16, 24)

    x0 = jnp.zeros_like(e)
    x1 = e + ks1

    def rounds(x0, x1, rots):
        for r in rots:
            x0 = x0 + x1
            x1 = ((x1 << jnp.uint32(r)) | (x1 >> jnp.uint32(32 - r))) ^ x0
        return x0, x1

    x0, x1 = rounds(x0, x1, rot1)
    x0 = x0 + ks1
    x1 = x1 + (ks2 + jnp.uint32(1))
    x0, x1 = rounds(x0, x1, rot2)
    x0 = x0 + ks2
    x1 = x1 + (ks0 + jnp.uint32(2))
    x0, x1 = rounds(x0, x1, rot1)
    x0 = x0 + ks0
    x1 = x1 + (ks1 + jnp.uint32(3))
    x0, x1 = rounds(x0, x1, rot2)
    x0 = x0 + ks1
    x1 = x1 + (ks2 + jnp.uint32(4))
    x0, x1 = rounds(x0, x1, rot1)
    x0 = x0 + ks2
    x1 = x1 + (ks0 + jnp.uint32(5))
    return x0 ^ x1


def _body(m_ref, x_ref, xt_ref, ign_ref):
    blk = pl.program_id(0)
    r = x_ref.shape[0]
    n = x_ref.shape[1]
    row = jnp.uint32(blk * r) + jax.lax.broadcasted_iota(jnp.uint32, (r, n), 0)
    col = jax.lax.broadcasted_iota(jnp.uint32, (r, n), 1)
    e = row * jnp.uint32(n) + col
    bits = _threefry_bits(e)
    mask = (bits >> jnp.uint32(9)) < m_ref[...].astype(jnp.uint32)

    x = x_ref[...]
    xt_ref[...] = jnp.where(mask, jnp.int32(21), x)
    ign_ref[...] = jnp.where(mask, x, jnp.int32(-1))


@functools.partial(jax.jit)
def kernel(x_0, t):
    b, n = x_0.shape
    x32 = x_0.astype(jnp.int32)
    p = t.astype(jnp.float32) / _TIMESTEPS
    m = jnp.ceil(p.astype(jnp.float64) * (2.0 ** 23)).astype(jnp.int32)
    m = m.reshape(b, 1)

    r = _ROWS_PER_BLOCK
    grid = (b // r,)
    _imap = lambda i: (i, np.int32(0))
    xt32, ign32 = pl.pallas_call(
        _body,
        grid=grid,
        in_specs=[
            pl.BlockSpec((r, 1), _imap),
            pl.BlockSpec((r, n), _imap),
        ],
        out_specs=[
            pl.BlockSpec((r, n), _imap),
            pl.BlockSpec((r, n), _imap),
        ],
        out_shape=[
            jax.ShapeDtypeStruct((b, n), jnp.int32),
            jax.ShapeDtypeStruct((b, n), jnp.int32),
        ],
        compiler_params=pltpu.CompilerParams(
            dimension_semantics=("parallel",),
        ),
    )(m, x32)
    xt32 = jnp.minimum(xt32, jnp.int32(21))
    ign32 = jnp.maximum(ign32, jnp.int32(-1))
    return (xt32.astype(jnp.int64), ign32.astype(jnp.int64))


# speedup vs baseline: 36.9968x; 1.0029x over previous
import functools

import jax
import jax.numpy as jnp
import numpy as np
from jax.experimental import pallas as pl
from jax.experimental.pallas import tpu as pltpu

_TIMESTEPS = 100
_RES = 200
_ROWS_PER_BLOCK = 256


def _threefry_bits(e):
    ks0 = jnp.uint32(0)
    ks1 = jnp.uint32(42)
    ks2 = jnp.uint32(0x1BD11BDA ^ 42)
    rot1 = (13, 15, 26, 6)
    rot2 = (17, 29, 16, 24)

    x1 = e + ks1

    def rounds(x0, x1, rots):
        for r in rots:
            x0 = x0 + x1
            x1 = ((x1 << jnp.uint32(r)) | (x1 >> jnp.uint32(32 - r))) ^ x0
        return x0, x1

    x0 = x1
    x1 = ((x1 << jnp.uint32(13)) | (x1 >> jnp.uint32(19))) ^ x0
    x0, x1 = rounds(x0, x1, rot1[1:])
    x0 = x0 + ks1
    x1 = x1 + (ks2 + jnp.uint32(1))
    x0, x1 = rounds(x0, x1, rot2)
    x0 = x0 + ks2
    x1 = x1 + (ks0 + jnp.uint32(2))
    x0, x1 = rounds(x0, x1, rot1)
    x0 = x0 + ks0
    x1 = x1 + (ks1 + jnp.uint32(3))
    x0, x1 = rounds(x0, x1, rot2)
    x0 = x0 + ks1
    x1 = x1 + (ks2 + jnp.uint32(4))
    x0, x1 = rounds(x0, x1, rot1)
    x0 = x0 + ks2
    x1 = x1 + (ks0 + jnp.uint32(5))
    return x0 ^ x1


def _body(m_ref, x_ref, xt_ref, ign_ref):
    blk = pl.program_id(0)
    r = x_ref.shape[0]
    n = x_ref.shape[1]
    row = jnp.uint32(blk * r) + jax.lax.broadcasted_iota(jnp.uint32, (r, n), 0)
    col = jax.lax.broadcasted_iota(jnp.uint32, (r, n), 1)
    e = row * jnp.uint32(n) + col
    bits = _threefry_bits(e)
    mask = (bits >> jnp.uint32(9)) < m_ref[...].astype(jnp.uint32)

    x = x_ref[...]
    xt_ref[...] = jnp.where(mask, jnp.int32(21), x)
    ign_ref[...] = jnp.where(mask, x, jnp.int32(-1))


@functools.partial(jax.jit)
def kernel(x_0, t):
    b, n = x_0.shape
    x32 = x_0.astype(jnp.int32)
    p = t.astype(jnp.float32) / _TIMESTEPS
    m = jnp.ceil(p.astype(jnp.float64) * (2.0 ** 23)).astype(jnp.int32)
    m = m.reshape(b, 1)

    r = _ROWS_PER_BLOCK
    grid = (b // r,)
    _imap = lambda i: (i, np.int32(0))
    xt32, ign32 = pl.pallas_call(
        _body,
        grid=grid,
        in_specs=[
            pl.BlockSpec((r, 1), _imap),
            pl.BlockSpec((r, n), _imap),
        ],
        out_specs=[
            pl.BlockSpec((r, n), _imap),
            pl.BlockSpec((r, n), _imap),
        ],
        out_shape=[
            jax.ShapeDtypeStruct((b, n), jnp.int32),
            jax.ShapeDtypeStruct((b, n), jnp.int32),
        ],
        compiler_params=pltpu.CompilerParams(
            dimension_semantics=("parallel",),
        ),
    )(m, x32)
    return (xt32.astype(jnp.int64), ign32.astype(jnp.int64))


# speedup vs baseline: 37.2198x; 1.0060x over previous
import functools

import jax
import jax.numpy as jnp
import numpy as np
from jax.experimental import pallas as pl
from jax.experimental.pallas import tpu as pltpu

_TIMESTEPS = 100
_RES = 200
_ROWS_PER_BLOCK = 256


def _threefry_bits(e):
    ks0 = jnp.uint32(0)
    ks1 = jnp.uint32(42)
    ks2 = jnp.uint32(0x1BD11BDA ^ 42)
    rot1 = (13, 15, 26, 6)
    rot2 = (17, 29, 16, 24)

    x1 = e + ks1

    def rounds(x0, x1, rots):
        for r in rots:
            x0 = x0 + x1
            x1 = ((x1 << jnp.uint32(r)) | (x1 >> jnp.uint32(32 - r))) ^ x0
        return x0, x1

    x0 = x1
    x1 = ((x1 << jnp.uint32(13)) | (x1 >> jnp.uint32(19))) ^ x0
    x0, x1 = rounds(x0, x1, rot1[1:])
    x0 = x0 + ks1
    x1 = x1 + (ks2 + jnp.uint32(1))
    x0, x1 = rounds(x0, x1, rot2)
    x0 = x0 + ks2
    x1 = x1 + (ks0 + jnp.uint32(2))
    x0, x1 = rounds(x0, x1, rot1)
    x0 = x0 + ks0
    x1 = x1 + (ks1 + jnp.uint32(3))
    x0, x1 = rounds(x0, x1, rot2)
    x0 = x0 + ks1
    x1 = x1 + (ks2 + jnp.uint32(4))
    x0, x1 = rounds(x0, x1, rot1)
    x0 = x0 + ks2
    x1 = x1 + (ks0 + jnp.uint32(5))
    return x0 ^ x1


def _body(m_ref, x_ref, xt_ref, ign_ref):
    blk = pl.program_id(0)
    r = x_ref.shape[0]
    n = x_ref.shape[1]
    row = jnp.uint32(blk * r) + jax.lax.broadcasted_iota(jnp.uint32, (r, n), 0)
    col = jax.lax.broadcasted_iota(jnp.uint32, (r, n), 1)
    e = row * jnp.uint32(n) + col
    bits = _threefry_bits(e)
    mask = (bits >> jnp.uint32(9)) < m_ref[...].astype(jnp.uint32)

    x = x_ref[...].astype(jnp.int8)
    xt_ref[...] = jnp.where(mask, jnp.int8(21), x)
    ign_ref[...] = jnp.where(mask, x, jnp.int8(-1))


@functools.partial(jax.jit)
def kernel(x_0, t):
    b, n = x_0.shape
    x32 = x_0.astype(jnp.int32)
    p = t.astype(jnp.float32) / _TIMESTEPS
    m = jnp.ceil(p.astype(jnp.float64) * (2.0 ** 23)).astype(jnp.int32)
    m = m.reshape(b, 1)

    r = _ROWS_PER_BLOCK
    grid = (b // r,)
    _imap = lambda i: (i, np.int32(0))
    xt32, ign32 = pl.pallas_call(
        _body,
        grid=grid,
        in_specs=[
            pl.BlockSpec((r, 1), _imap),
            pl.BlockSpec((r, n), _imap),
        ],
        out_specs=[
            pl.BlockSpec((r, n), _imap),
            pl.BlockSpec((r, n), _imap),
        ],
        out_shape=[
            jax.ShapeDtypeStruct((b, n), jnp.int8),
            jax.ShapeDtypeStruct((b, n), jnp.int8),
        ],
        compiler_params=pltpu.CompilerParams(
            dimension_semantics=("parallel",),
        ),
    )(m, x32)
    return (xt32.astype(jnp.int64), ign32.astype(jnp.int64))


# speedup vs baseline: 37.4730x; 1.0068x over previous
import functools

import jax
import jax.numpy as jnp
import numpy as np
from jax.experimental import pallas as pl
from jax.experimental.pallas import tpu as pltpu

_TIMESTEPS = 100
_RES = 200
_ROWS_PER_BLOCK = 256


def _threefry_bits(e):
    ks0 = jnp.uint32(0)
    ks1 = jnp.uint32(42)
    ks2 = jnp.uint32(0x1BD11BDA ^ 42)
    rot1 = (13, 15, 26, 6)
    rot2 = (17, 29, 16, 24)

    x1 = e + ks1

    def rounds(x0, x1, rots):
        for r in rots:
            x0 = x0 + x1
            x1 = ((x1 << jnp.uint32(r)) | (x1 >> jnp.uint32(32 - r))) ^ x0
        return x0, x1

    x0 = x1
    x1 = ((x1 << jnp.uint32(13)) | (x1 >> jnp.uint32(19))) ^ x0
    x0, x1 = rounds(x0, x1, rot1[1:])
    x0 = x0 + ks1
    x1 = x1 + (ks2 + jnp.uint32(1))
    x0, x1 = rounds(x0, x1, rot2)
    x0 = x0 + ks2
    x1 = x1 + (ks0 + jnp.uint32(2))
    x0, x1 = rounds(x0, x1, rot1)
    x0 = x0 + ks0
    x1 = x1 + (ks1 + jnp.uint32(3))
    x0, x1 = rounds(x0, x1, rot2)
    x0 = x0 + ks1
    x1 = x1 + (ks2 + jnp.uint32(4))
    x0, x1 = rounds(x0, x1, rot1)
    x0 = x0 + ks2
    x1 = x1 + (ks0 + jnp.uint32(5))
    return x0 ^ x1


def _body(m_ref, x_ref, xt_ref, ign_ref):
    blk = pl.program_id(0)
    r = x_ref.shape[0]
    n = x_ref.shape[1]
    row = jnp.uint32(blk * r) + jax.lax.broadcasted_iota(jnp.uint32, (r, n), 0)
    col = jax.lax.broadcasted_iota(jnp.uint32, (r, n), 1)
    e = row * jnp.uint32(n) + col
    bits = _threefry_bits(e)
    mask = (bits >> jnp.uint32(9)) < m_ref[...].astype(jnp.uint32)

    x = x_ref[...]
    xt_ref[...] = jnp.where(mask, jnp.int8(21), x)
    ign_ref[...] = jnp.where(mask, x, jnp.int8(-1))


@functools.partial(jax.jit)
def kernel(x_0, t):
    b, n = x_0.shape
    x32 = x_0.astype(jnp.int8)
    p = t.astype(jnp.float32) / _TIMESTEPS
    m = jnp.ceil(p.astype(jnp.float64) * (2.0 ** 23)).astype(jnp.int32)
    m = m.reshape(b, 1)

    r = _ROWS_PER_BLOCK
    grid = (b // r,)
    _imap = lambda i: (i, np.int32(0))
    xt32, ign32 = pl.pallas_call(
        _body,
        grid=grid,
        in_specs=[
            pl.BlockSpec((r, 1), _imap),
            pl.BlockSpec((r, n), _imap),
        ],
        out_specs=[
            pl.BlockSpec((r, n), _imap),
            pl.BlockSpec((r, n), _imap),
        ],
        out_shape=[
            jax.ShapeDtypeStruct((b, n), jnp.int8),
            jax.ShapeDtypeStruct((b, n), jnp.int8),
        ],
        compiler_params=pltpu.CompilerParams(
            dimension_semantics=("parallel",),
        ),
    )(m, x32)
    return (xt32.astype(jnp.int64), ign32.astype(jnp.int64))
